# Initial kernel scaffold; baseline (speedup 1.0000x reference)
import jax
import jax.numpy as jnp
from jax.experimental import pallas as pl


def kernel(inputs, adj, mask_item, item, adj_all, num, embedding, a0, a1, a2, a3, gw1, gw2, gw3):
    raise NotImplementedError("write your pallas kernel here")


# speedup vs baseline: 1.2844x; 1.2844x over previous
import functools

import jax
import jax.numpy as jnp
from jax import lax
from jax.experimental import pallas as pl
from jax.experimental.pallas import tpu as pltpu
from jax.experimental.pallas import tpu_sc as plsc

DIM = 128
NC, NS = 2, 16
NW = NC * NS

_MESH = dict(core_axis_name="c", subcore_axis_name="s")


def _wid():
    return lax.axis_index("s") * NC + lax.axis_index("c")


def _pad_to(x, n):
    return jnp.concatenate([x, jnp.zeros((n - x.shape[0],), x.dtype)])


def _sc_chase(adj_flat, num_flat, idx2, s):
    nw, npt = idx2.shape
    ngroups = npt // 16
    nblocks = adj_flat.shape[0]
    n_rows = nw * npt

    def body(adj_hbm, num_hbm, idx_hbm, oa_hbm, on_hbm,
             idx_v, bidx_v, ablk, nblk, oa_v, on_v, sem):
        wid = _wid()
        pltpu.sync_copy(idx_hbm.at[wid], idx_v)
        lane = lax.iota(jnp.int32, 16)

        def group(g, carry):
            t16 = idx_v[pl.ds(g * 16, 16)]
            w = t16 * s
            b0 = lax.shift_right_logical(w, 7)
            b1 = jnp.minimum(b0 + 1, nblocks - 1)
            o = lax.bitwise_and(w, 127)
            bidx_v[pl.ds(0, 16)] = b0
            bidx_v[pl.ds(16, 16)] = b1
            pltpu.async_copy(adj_hbm.at[bidx_v], ablk, sem).wait()
            pltpu.async_copy(num_hbm.at[bidx_v], nblk, sem).wait()
            for k in range(s):
                ck = o + k
                row = lane + 16 * lax.shift_right_logical(ck, 7)
                col = lax.bitwise_and(ck, 127)
                va = plsc.load_gather(ablk, [row, col])
                vn = plsc.load_gather(nblk, [row, col])
                kv = jnp.full((16,), k, jnp.int32)
                plsc.store_scatter(oa_v, [lane, kv], va)
                plsc.store_scatter(on_v, [lane, kv], vn)
            base = (wid * ngroups + g) * 16
            pltpu.sync_copy(oa_v, oa_hbm.at[pl.ds(base, 16)])
            pltpu.sync_copy(on_v, on_hbm.at[pl.ds(base, 16)])
            return carry

        lax.fori_loop(0, ngroups, group, 0)

    f = pl.kernel(
        body,
        out_type=(jax.ShapeDtypeStruct((n_rows, 16), jnp.int32),
                  jax.ShapeDtypeStruct((n_rows, 16), jnp.float32)),
        mesh=plsc.VectorSubcoreMesh(**_MESH),
        scratch_types=[pltpu.VMEM((ngroups * 16,), jnp.int32),
                       pltpu.VMEM((32,), jnp.int32),
                       pltpu.VMEM((32, 128), jnp.int32),
                       pltpu.VMEM((32, 128), jnp.float32),
                       pltpu.VMEM((16, 16), jnp.int32),
                       pltpu.VMEM((16, 16), jnp.float32),
                       pltpu.SemaphoreType.DMA],
        compiler_params=pltpu.CompilerParams(needs_layout_passes=False),
    )
    return f(adj_flat, num_flat, idx2)


def _sc_embed_gather(table, idx3):
    nw, nchunks, c = idx3.shape
    d = table.shape[1]
    n_rows = nw * nchunks * c
    assert nchunks % 2 == 0

    def body(tab_hbm, idx_hbm, out_hbm, idx_v, buf0, buf1, sem0, sem1):
        wid = _wid()
        pltpu.sync_copy(idx_hbm.at[wid], idx_v)

        def step(j, carry):
            i0 = 2 * j
            i1 = i0 + 1
            cp0 = pltpu.async_copy(tab_hbm.at[idx_v.at[i0]], buf0, sem0)
            cp1 = pltpu.async_copy(tab_hbm.at[idx_v.at[i1]], buf1, sem1)
            cp0.wait()
            pltpu.sync_copy(buf0, out_hbm.at[pl.ds((wid * nchunks + i0) * c, c)])
            cp1.wait()
            pltpu.sync_copy(buf1, out_hbm.at[pl.ds((wid * nchunks + i1) * c, c)])
            return carry

        lax.fori_loop(0, nchunks // 2, step, 0)

    f = pl.kernel(
        body,
        out_type=jax.ShapeDtypeStruct((n_rows, d), jnp.float32),
        mesh=plsc.VectorSubcoreMesh(**_MESH),
        scratch_types=[pltpu.VMEM((nchunks, c), jnp.int32),
                       pltpu.VMEM((c, d), jnp.float32),
                       pltpu.VMEM((c, d), jnp.float32),
                       pltpu.SemaphoreType.DMA,
                       pltpu.SemaphoreType.DMA],
    )
    return f(table, idx3)


def _tc_local(h, item_emb, adj, maskf, a_cat):
    b, l, d = h.shape

    def body(h_ref, it_ref, adj_ref, m_ref, a_ref, hl_ref, si_ref):
        hh = h_ref[0]
        aa = a_ref[...]
        ad = adj_ref[0]
        alpha = jnp.full((l, l), -9e15, dtype=jnp.float32)
        for k in range(4):
            hk = hh * aa[k][None, :]
            ek = lax.dot_general(hk, hh, (((1,), (1,)), ((), ())),
                                 preferred_element_type=jnp.float32)
            ek = jnp.where(ek >= 0, ek, 0.2 * ek)
            alpha = jnp.where(ad == (k + 1), ek, alpha)
        alpha = jax.nn.softmax(alpha, axis=-1)
        hl_ref[0] = jnp.dot(alpha, hh, preferred_element_type=jnp.float32)
        m = m_ref[0, 0]
        it = it_ref[0] * m[:, None]
        si_ref[0, 0] = jnp.sum(it, axis=0) / jnp.sum(m)

    return pl.pallas_call(
        body,
        grid=(b,),
        in_specs=[
            pl.BlockSpec((1, l, d), lambda i: (i, 0, 0)),
            pl.BlockSpec((1, l, d), lambda i: (i, 0, 0)),
            pl.BlockSpec((1, l, l), lambda i: (i, 0, 0)),
            pl.BlockSpec((1, 1, l), lambda i: (i, 0, 0)),
            pl.BlockSpec((4, d), lambda i: (0, 0)),
        ],
        out_specs=[
            pl.BlockSpec((1, l, d), lambda i: (i, 0, 0)),
            pl.BlockSpec((1, 1, d), lambda i: (i, 0, 0)),
        ],
        out_shape=[jax.ShapeDtypeStruct((b, l, d), jnp.float32),
                   jax.ShapeDtypeStruct((b, 1, d), jnp.float32)],
    )(h, item_emb, adj, maskf, a_cat)


def _tc_global(self_v, neigh_rows, w, si, w1a, w1b, w2r, w3a, w3b, mc,
               resid=None):
    b, m, d = self_v.shape
    s = w.shape[2]
    nblk = m // mc

    def body(*refs):
        if resid is not None:
            (sf_ref, nb_ref, w_ref, si_ref, w1a_ref, w1b_ref, w2_ref,
             w3a_ref, w3b_ref, res_ref, out_ref) = refs
        else:
            (sf_ref, nb_ref, w_ref, si_ref, w1a_ref, w1b_ref, w2_ref,
             w3a_ref, w3b_ref, out_ref) = refs
            res_ref = None
        nb = nb_ref[...]
        sv = si_ref[0, 0]
        x = nb * sv[None, :]
        t = jnp.dot(x, w1a_ref[...], preferred_element_type=jnp.float32)
        t3 = t.reshape(mc, s, d) + w_ref[0][:, :, None] * w1b_ref[...][0][None, None, :]
        t3 = jnp.where(t3 >= 0, t3, 0.2 * t3)
        sc = jnp.sum(t3 * w2_ref[...][0][None, None, :], axis=-1)
        p = jax.nn.softmax(sc, axis=-1)
        neigh = jnp.sum(p[:, :, None] * nb.reshape(mc, s, d), axis=1)
        o = jnp.dot(sf_ref[0], w3a_ref[...], preferred_element_type=jnp.float32)
        o = o + jnp.dot(neigh, w3b_ref[...], preferred_element_type=jnp.float32)
        o = jnp.maximum(o, 0.0)
        if res_ref is not None:
            o = o + res_ref[0]
        out_ref[0] = o

    in_specs = [
        pl.BlockSpec((1, mc, d), lambda i, j: (i, j, 0)),
        pl.BlockSpec((mc * s, d), lambda i, j, _n=nblk: (i * _n + j, 0)),
        pl.BlockSpec((1, mc, s), lambda i, j: (i, j, 0)),
        pl.BlockSpec((1, 1, d), lambda i, j: (i, 0, 0)),
        pl.BlockSpec((d, d), lambda i, j: (0, 0)),
        pl.BlockSpec((1, d), lambda i, j: (0, 0)),
        pl.BlockSpec((1, d), lambda i, j: (0, 0)),
        pl.BlockSpec((d, d), lambda i, j: (0, 0)),
        pl.BlockSpec((d, d), lambda i, j: (0, 0)),
    ]
    args = [self_v, neigh_rows, w, si, w1a, w1b, w2r, w3a, w3b]
    if resid is not None:
        in_specs.append(pl.BlockSpec((1, mc, d), lambda i, j: (i, j, 0)))
        args.append(resid)

    return pl.pallas_call(
        body,
        grid=(b, nblk),
        in_specs=in_specs,
        out_specs=pl.BlockSpec((1, mc, d), lambda i, j: (i, j, 0)),
        out_shape=jax.ShapeDtypeStruct((b, m, d), jnp.float32),
    )(*args)


def kernel(inputs, adj, mask_item, item, adj_all, num, embedding,
           a0, a1, a2, a3, gw1, gw2, gw3):
    b, l = inputs.shape
    s = adj_all.shape[1]
    d = embedding.shape[1]

    adj_flat = adj_all.reshape(-1, 128)
    num_flat = num.reshape(-1, 128)

    t0 = inputs.reshape(-1)
    t0p = _pad_to(t0, NW * 64).reshape(NW, 64)
    a1_rows, n1_rows = _sc_chase(adj_flat, num_flat, t0p, s)

    n_t1 = b * l * s
    t1 = a1_rows[:b * l, :s].reshape(-1)
    w1n = n1_rows[:b * l, :s].reshape(b, l, s)
    t1p = _pad_to(t1, NW * 640).reshape(NW, 640)
    a2_rows, n2_rows = _sc_chase(adj_flat, num_flat, t1p, s)

    t2 = a2_rows[:n_t1, :s].reshape(-1)
    w2n = n2_rows[:n_t1, :s].reshape(b, l * s, s)
    t2p = _pad_to(t2, NW * 58 * 128).reshape(NW, 58, 128)

    e2buf = _sc_embed_gather(embedding, t2p)

    small_idx = jnp.concatenate([t0, item.reshape(-1), t1])
    smp = _pad_to(small_idx, NW * 6 * 128).reshape(NW, 6, 128)
    small = _sc_embed_gather(embedding, smp)

    h = small[:b * l].reshape(b, l, d)
    item_emb = small[b * l:2 * b * l].reshape(b, l, d)
    e1 = small[2 * b * l:2 * b * l + n_t1]

    a_cat = jnp.concatenate([a0, a1, a2, a3], axis=1).T
    maskf = mask_item.astype(jnp.float32).reshape(b, 1, l)
    h_local, si = _tc_local(h, item_emb, adj, maskf, a_cat)

    def wsplit(k):
        return (gw1[k, :d], gw1[k, d:].reshape(1, d), gw2[k].reshape(1, d),
                gw3[k, :d], gw3[k, d:])

    w1a0, w1b0, w2r0, w3a0, w3b0 = wsplit(0)
    w1a1, w1b1, w2r1, w3a1, w3b1 = wsplit(1)

    v1 = _tc_global(e1.reshape(b, l * s, d), e2buf, w2n, si,
                    w1a0, w1b0, w2r0, w3a0, w3b0, mc=120)
    v0 = _tc_global(h, e1, w1n, si,
                    w1a0, w1b0, w2r0, w3a0, w3b0, mc=l)
    out = _tc_global(v0, v1.reshape(b * l * s, d), w1n, si,
                     w1a1, w1b1, w2r1, w3a1, w3b1, mc=l,
                     resid=h_local)
    return out


# speedup vs baseline: 1.9342x; 1.5059x over previous
import functools

import jax
import jax.numpy as jnp
from jax import lax
from jax.experimental import pallas as pl
from jax.experimental.pallas import tpu as pltpu
from jax.experimental.pallas import tpu_sc as plsc

DIM = 128
NC, NS = 2, 16
NW = NC * NS

_MESH = dict(core_axis_name="c", subcore_axis_name="s")


def _wid():
    return lax.axis_index("s") * NC + lax.axis_index("c")


def _pad_to(x, n):
    return jnp.concatenate([x, jnp.zeros((n - x.shape[0],), x.dtype)])


def _sc_chase(adj_flat, num_flat, idx2, s):
    nw, npt = idx2.shape
    c = 64
    ngroups = npt // c
    assert npt % c == 0 and (ngroups == 1 or ngroups % 2 == 0)
    nblocks = adj_flat.shape[0]
    n_rows = nw * npt

    def body(adj_hbm, num_hbm, idx_hbm, oa_hbm, on_hbm,
             idx_v, bidx0, bidx1, ablk0, nblk0, ablk1, nblk1,
             oa_v, on_v, sema0, semn0, sema1, semn1):
        wid = _wid()
        pltpu.sync_copy(idx_hbm.at[wid], idx_v)
        lane = lax.iota(jnp.int32, 16)
        bufs = ((bidx0, ablk0, nblk0, sema0, semn0),
                (bidx1, ablk1, nblk1, sema1, semn1))

        def issue(g, par):
            bidx, ablk, nblk, sema, semn = bufs[par]
            for q in range(c // 16):
                t16 = idx_v[pl.ds(g * c + q * 16, 16)]
                w = t16 * s
                b0 = lax.shift_right_logical(w, 7)
                bidx[pl.ds(32 * q, 16)] = b0
                bidx[pl.ds(32 * q + 16, 16)] = jnp.minimum(b0 + 1, nblocks - 1)
            cpa = pltpu.async_copy(adj_hbm.at[bidx], ablk, sema)
            cpn = pltpu.async_copy(num_hbm.at[bidx], nblk, semn)
            return cpa, cpn

        def extract(g, par, cps):
            _, ablk, nblk, _, _ = bufs[par]
            cps[0].wait()
            cps[1].wait()
            for q in range(c // 16):
                t16 = idx_v[pl.ds(g * c + q * 16, 16)]
                o = lax.bitwise_and(t16 * s, 127)
                for k in range(s):
                    ck = o + k
                    row = 32 * q + lane + 16 * lax.shift_right_logical(ck, 7)
                    col = lax.bitwise_and(ck, 127)
                    va = plsc.load_gather(ablk, [row, col])
                    vn = plsc.load_gather(nblk, [row, col])
                    orow = lane + 16 * q
                    kv = jnp.full((16,), k, jnp.int32)
                    plsc.store_scatter(oa_v, [orow, kv], va)
                    plsc.store_scatter(on_v, [orow, kv], vn)
            base = (wid * ngroups + g) * c
            pltpu.sync_copy(oa_v, oa_hbm.at[pl.ds(base, c)])
            pltpu.sync_copy(on_v, on_hbm.at[pl.ds(base, c)])

        if ngroups == 1:
            extract(0, 0, issue(0, 0))
        else:
            cps0 = issue(0, 0)
            cps1 = issue(1, 1)

            def round2(g2, carry):
                g = 2 * g2
                extract(g, 0, cps0)

                @pl.when(g + 2 < ngroups)
                def _():
                    issue(g + 2, 0)
                extract(g + 1, 1, cps1)

                @pl.when(g + 3 < ngroups)
                def _():
                    issue(g + 3, 1)
                return carry

            lax.fori_loop(0, ngroups // 2, round2, 0)

    f = pl.kernel(
        body,
        out_type=(jax.ShapeDtypeStruct((n_rows, 16), jnp.int32),
                  jax.ShapeDtypeStruct((n_rows, 16), jnp.float32)),
        mesh=plsc.VectorSubcoreMesh(**_MESH),
        scratch_types=[pltpu.VMEM((npt,), jnp.int32),
                       pltpu.VMEM((2 * c,), jnp.int32),
                       pltpu.VMEM((2 * c,), jnp.int32),
                       pltpu.VMEM((2 * c, 128), jnp.int32),
                       pltpu.VMEM((2 * c, 128), jnp.float32),
                       pltpu.VMEM((2 * c, 128), jnp.int32),
                       pltpu.VMEM((2 * c, 128), jnp.float32),
                       pltpu.VMEM((c, 16), jnp.int32),
                       pltpu.VMEM((c, 16), jnp.float32),
                       pltpu.SemaphoreType.DMA,
                       pltpu.SemaphoreType.DMA,
                       pltpu.SemaphoreType.DMA,
                       pltpu.SemaphoreType.DMA],
        compiler_params=pltpu.CompilerParams(needs_layout_passes=False),
    )
    return f(adj_flat, num_flat, idx2)


def _sc_embed_gather(table, idx3, nbuf):
    nw, nchunks, c = idx3.shape
    d = table.shape[1]
    n_rows = nw * nchunks * c
    assert nchunks % nbuf == 0

    def body(tab_hbm, idx_hbm, out_hbm, idx_v, *rest):
        bufs = rest[:nbuf]
        gsems = rest[nbuf:2 * nbuf]
        ssems = rest[2 * nbuf:3 * nbuf]
        wid = _wid()
        pltpu.sync_copy(idx_hbm.at[wid], idx_v)

        def gather(i, r):
            pltpu.async_copy(tab_hbm.at[idx_v.at[i]], bufs[r], gsems[r])

        def store(i, r):
            pltpu.async_copy(
                bufs[r], out_hbm.at[pl.ds((wid * nchunks + i) * c, c)],
                ssems[r])

        def g_wait(r):
            pltpu.make_async_copy(tab_hbm.at[idx_v.at[0]], bufs[r],
                                  gsems[r]).wait()

        def s_wait(r):
            pltpu.make_async_copy(bufs[r], out_hbm.at[pl.ds(0, c)],
                                  ssems[r]).wait()

        for r in range(nbuf):
            gather(r, r)

        def round_(j, carry):
            for r in range(nbuf):
                i = j * nbuf + r
                g_wait(r)
                store(i, r)

                @pl.when(i + nbuf < nchunks)
                def _():
                    s_wait(r)
                    gather(i + nbuf, r)
            return carry

        lax.fori_loop(0, nchunks // nbuf, round_, 0)
        for r in range(nbuf):
            s_wait(r)

    scratch = ([pltpu.VMEM((nchunks, c), jnp.int32)]
               + [pltpu.VMEM((c, d), jnp.float32) for _ in range(nbuf)]
               + [pltpu.SemaphoreType.DMA for _ in range(2 * nbuf)])
    f = pl.kernel(
        body,
        out_type=jax.ShapeDtypeStruct((n_rows, d), jnp.float32),
        mesh=plsc.VectorSubcoreMesh(**_MESH),
        scratch_types=scratch,
    )
    return f(table, idx3)


def _tc_local(h, item_emb, adj, maskf, a_cat):
    b, l, d = h.shape

    def body(h_ref, it_ref, adj_ref, m_ref, a_ref, hl_ref, si_ref):
        hh = h_ref[0]
        aa = a_ref[...]
        ad = adj_ref[0]
        alpha = jnp.full((l, l), -9e15, dtype=jnp.float32)
        for k in range(4):
            hk = hh * aa[k][None, :]
            ek = lax.dot_general(hk, hh, (((1,), (1,)), ((), ())),
                                 preferred_element_type=jnp.float32)
            ek = jnp.where(ek >= 0, ek, 0.2 * ek)
            alpha = jnp.where(ad == (k + 1), ek, alpha)
        alpha = jax.nn.softmax(alpha, axis=-1)
        hl_ref[0] = jnp.dot(alpha, hh, preferred_element_type=jnp.float32)
        m = m_ref[0, 0]
        it = it_ref[0] * m[:, None]
        si_ref[0, 0] = jnp.sum(it, axis=0) / jnp.sum(m)

    return pl.pallas_call(
        body,
        grid=(b,),
        in_specs=[
            pl.BlockSpec((1, l, d), lambda i: (i, 0, 0)),
            pl.BlockSpec((1, l, d), lambda i: (i, 0, 0)),
            pl.BlockSpec((1, l, l), lambda i: (i, 0, 0)),
            pl.BlockSpec((1, 1, l), lambda i: (i, 0, 0)),
            pl.BlockSpec((4, d), lambda i: (0, 0)),
        ],
        out_specs=[
            pl.BlockSpec((1, l, d), lambda i: (i, 0, 0)),
            pl.BlockSpec((1, 1, d), lambda i: (i, 0, 0)),
        ],
        out_shape=[jax.ShapeDtypeStruct((b, l, d), jnp.float32),
                   jax.ShapeDtypeStruct((b, 1, d), jnp.float32)],
    )(h, item_emb, adj, maskf, a_cat)


def _tc_global(self_v, neigh_rows, w, si, w1a, w1b, w2r, w3a, w3b, mc,
               resid=None):
    b, m, d = self_v.shape
    s = w.shape[2]
    nblk = m // mc

    def body(*refs):
        if resid is not None:
            (sf_ref, nb_ref, w_ref, si_ref, w1a_ref, w1b_ref, w2_ref,
             w3a_ref, w3b_ref, res_ref, out_ref) = refs
        else:
            (sf_ref, nb_ref, w_ref, si_ref, w1a_ref, w1b_ref, w2_ref,
             w3a_ref, w3b_ref, out_ref) = refs
            res_ref = None
        nb = nb_ref[...]
        sv = si_ref[0, 0]
        x = nb * sv[None, :]
        t = jnp.dot(x, w1a_ref[...], preferred_element_type=jnp.float32)
        t3 = t.reshape(mc, s, d) + w_ref[0][:, :, None] * w1b_ref[...][0][None, None, :]
        t3 = jnp.where(t3 >= 0, t3, 0.2 * t3)
        sc = jnp.sum(t3 * w2_ref[...][0][None, None, :], axis=-1)
        p = jax.nn.softmax(sc, axis=-1)
        neigh = jnp.sum(p[:, :, None] * nb.reshape(mc, s, d), axis=1)
        o = jnp.dot(sf_ref[0], w3a_ref[...], preferred_element_type=jnp.float32)
        o = o + jnp.dot(neigh, w3b_ref[...], preferred_element_type=jnp.float32)
        o = jnp.maximum(o, 0.0)
        if res_ref is not None:
            o = o + res_ref[0]
        out_ref[0] = o

    in_specs = [
        pl.BlockSpec((1, mc, d), lambda i, j: (i, j, 0)),
        pl.BlockSpec((mc * s, d), lambda i, j, _n=nblk: (i * _n + j, 0)),
        pl.BlockSpec((1, mc, s), lambda i, j: (i, j, 0)),
        pl.BlockSpec((1, 1, d), lambda i, j: (i, 0, 0)),
        pl.BlockSpec((d, d), lambda i, j: (0, 0)),
        pl.BlockSpec((1, d), lambda i, j: (0, 0)),
        pl.BlockSpec((1, d), lambda i, j: (0, 0)),
        pl.BlockSpec((d, d), lambda i, j: (0, 0)),
        pl.BlockSpec((d, d), lambda i, j: (0, 0)),
    ]
    args = [self_v, neigh_rows, w, si, w1a, w1b, w2r, w3a, w3b]
    if resid is not None:
        in_specs.append(pl.BlockSpec((1, mc, d), lambda i, j: (i, j, 0)))
        args.append(resid)

    return pl.pallas_call(
        body,
        grid=(b, nblk),
        in_specs=in_specs,
        out_specs=pl.BlockSpec((1, mc, d), lambda i, j: (i, j, 0)),
        out_shape=jax.ShapeDtypeStruct((b, m, d), jnp.float32),
    )(*args)


def kernel(inputs, adj, mask_item, item, adj_all, num, embedding,
           a0, a1, a2, a3, gw1, gw2, gw3):
    b, l = inputs.shape
    s = adj_all.shape[1]
    d = embedding.shape[1]

    adj_flat = adj_all.reshape(-1, 128)
    num_flat = num.reshape(-1, 128)

    t0 = inputs.reshape(-1)
    t0p = _pad_to(t0, NW * 64).reshape(NW, 64)
    a1_rows, n1_rows = _sc_chase(adj_flat, num_flat, t0p, s)

    n_t1 = b * l * s
    t1 = a1_rows[:b * l, :s].reshape(-1)
    w1n = n1_rows[:b * l, :s].reshape(b, l, s)
    t1p = _pad_to(t1, NW * 640).reshape(NW, 640)
    a2_rows, n2_rows = _sc_chase(adj_flat, num_flat, t1p, s)

    t2 = a2_rows[:n_t1, :s].reshape(-1)
    w2n = n2_rows[:n_t1, :s].reshape(b, l * s, s)
    t2p = t2.reshape(NW, 75, 96)

    e2buf = _sc_embed_gather(embedding, t2p, nbuf=5)

    small_idx = jnp.concatenate([t0, item.reshape(-1), t1])
    smp = _pad_to(small_idx, NW * 8 * 96).reshape(NW, 8, 96)
    small = _sc_embed_gather(embedding, smp, nbuf=4)

    h = small[:b * l].reshape(b, l, d)
    item_emb = small[b * l:2 * b * l].reshape(b, l, d)
    e1 = small[2 * b * l:2 * b * l + n_t1]

    a_cat = jnp.concatenate([a0, a1, a2, a3], axis=1).T
    maskf = mask_item.astype(jnp.float32).reshape(b, 1, l)
    h_local, si = _tc_local(h, item_emb, adj, maskf, a_cat)

    def wsplit(k):
        return (gw1[k, :d], gw1[k, d:].reshape(1, d), gw2[k].reshape(1, d),
                gw3[k, :d], gw3[k, d:])

    w1a0, w1b0, w2r0, w3a0, w3b0 = wsplit(0)
    w1a1, w1b1, w2r1, w3a1, w3b1 = wsplit(1)

    v1 = _tc_global(e1.reshape(b, l * s, d), e2buf, w2n, si,
                    w1a0, w1b0, w2r0, w3a0, w3b0, mc=120)
    v0 = _tc_global(h, e1, w1n, si,
                    w1a0, w1b0, w2r0, w3a0, w3b0, mc=l)
    out = _tc_global(v0, v1.reshape(b * l * s, d), w1n, si,
                     w1a1, w1b1, w2r1, w3a1, w3b1, mc=l,
                     resid=h_local)
    return out


# speedup vs baseline: 1.9785x; 1.0229x over previous
import functools

import jax
import jax.numpy as jnp
from jax import lax
from jax.experimental import pallas as pl
from jax.experimental.pallas import tpu as pltpu
from jax.experimental.pallas import tpu_sc as plsc

DIM = 128
NC, NS = 2, 16
NW = NC * NS

_MESH = dict(core_axis_name="c", subcore_axis_name="s")


def _wid():
    return lax.axis_index("s") * NC + lax.axis_index("c")


def _pad_to(x, n):
    return jnp.concatenate([x, jnp.zeros((n - x.shape[0],), x.dtype)])


def _sc_chase(adj_flat, num_flat, idx2, s):
    nw, npt = idx2.shape
    c = 64
    ngroups = npt // c
    assert npt % c == 0 and (ngroups == 1 or ngroups % 2 == 0)
    nblocks = adj_flat.shape[0]
    n_rows = nw * npt

    def body(adj_hbm, num_hbm, idx_hbm, oa_hbm, on_hbm,
             idx_v, bidx0, bidx1, ablk0, nblk0, ablk1, nblk1,
             oa_v, on_v, sema0, semn0, sema1, semn1):
        wid = _wid()
        pltpu.sync_copy(idx_hbm.at[wid], idx_v)
        lane = lax.iota(jnp.int32, 16)
        bufs = ((bidx0, ablk0, nblk0, sema0, semn0),
                (bidx1, ablk1, nblk1, sema1, semn1))

        def issue(g, par):
            bidx, ablk, nblk, sema, semn = bufs[par]
            for q in range(c // 16):
                t16 = idx_v[pl.ds(g * c + q * 16, 16)]
                w = t16 * s
                b0 = lax.shift_right_logical(w, 7)
                bidx[pl.ds(32 * q, 16)] = b0
                bidx[pl.ds(32 * q + 16, 16)] = jnp.minimum(b0 + 1, nblocks - 1)
            cpa = pltpu.async_copy(adj_hbm.at[bidx], ablk, sema)
            cpn = pltpu.async_copy(num_hbm.at[bidx], nblk, semn)
            return cpa, cpn

        def extract(g, par, cps):
            _, ablk, nblk, _, _ = bufs[par]
            cps[0].wait()
            cps[1].wait()
            for q in range(c // 16):
                t16 = idx_v[pl.ds(g * c + q * 16, 16)]
                o = lax.bitwise_and(t16 * s, 127)
                for k in range(s):
                    ck = o + k
                    row = 32 * q + lane + 16 * lax.shift_right_logical(ck, 7)
                    col = lax.bitwise_and(ck, 127)
                    va = plsc.load_gather(ablk, [row, col])
                    vn = plsc.load_gather(nblk, [row, col])
                    orow = lane + 16 * q
                    kv = jnp.full((16,), k, jnp.int32)
                    plsc.store_scatter(oa_v, [orow, kv], va)
                    plsc.store_scatter(on_v, [orow, kv], vn)
            base = (wid * ngroups + g) * c
            pltpu.sync_copy(oa_v, oa_hbm.at[pl.ds(base, c)])
            pltpu.sync_copy(on_v, on_hbm.at[pl.ds(base, c)])

        if ngroups == 1:
            extract(0, 0, issue(0, 0))
        else:
            cps0 = issue(0, 0)
            cps1 = issue(1, 1)

            def round2(g2, carry):
                g = 2 * g2
                extract(g, 0, cps0)

                @pl.when(g + 2 < ngroups)
                def _():
                    issue(g + 2, 0)
                extract(g + 1, 1, cps1)

                @pl.when(g + 3 < ngroups)
                def _():
                    issue(g + 3, 1)
                return carry

            lax.fori_loop(0, ngroups // 2, round2, 0)

    f = pl.kernel(
        body,
        out_type=(jax.ShapeDtypeStruct((n_rows, 16), jnp.int32),
                  jax.ShapeDtypeStruct((n_rows, 16), jnp.float32)),
        mesh=plsc.VectorSubcoreMesh(**_MESH),
        scratch_types=[pltpu.VMEM((npt,), jnp.int32),
                       pltpu.VMEM((2 * c,), jnp.int32),
                       pltpu.VMEM((2 * c,), jnp.int32),
                       pltpu.VMEM((2 * c, 128), jnp.int32),
                       pltpu.VMEM((2 * c, 128), jnp.float32),
                       pltpu.VMEM((2 * c, 128), jnp.int32),
                       pltpu.VMEM((2 * c, 128), jnp.float32),
                       pltpu.VMEM((c, 16), jnp.int32),
                       pltpu.VMEM((c, 16), jnp.float32),
                       pltpu.SemaphoreType.DMA,
                       pltpu.SemaphoreType.DMA,
                       pltpu.SemaphoreType.DMA,
                       pltpu.SemaphoreType.DMA],
        compiler_params=pltpu.CompilerParams(needs_layout_passes=False),
    )
    return f(adj_flat, num_flat, idx2)


def _sc_embed_gather(table, idx3, nbuf):
    nw, nchunks, c = idx3.shape
    d = table.shape[1]
    n_rows = nw * nchunks * c
    assert nchunks % nbuf == 0

    def body(tab_hbm, idx_hbm, out_hbm, idx_v, *rest):
        bufs = rest[:nbuf]
        gsems = rest[nbuf:2 * nbuf]
        ssems = rest[2 * nbuf:3 * nbuf]
        wid = _wid()
        pltpu.sync_copy(idx_hbm.at[wid], idx_v)

        def gather(i, r):
            pltpu.async_copy(tab_hbm.at[idx_v.at[i]], bufs[r], gsems[r])

        def store(i, r):
            pltpu.async_copy(
                bufs[r], out_hbm.at[pl.ds((wid * nchunks + i) * c, c)],
                ssems[r])

        def g_wait(r):
            pltpu.make_async_copy(tab_hbm.at[idx_v.at[0]], bufs[r],
                                  gsems[r]).wait()

        def s_wait(r):
            pltpu.make_async_copy(bufs[r], out_hbm.at[pl.ds(0, c)],
                                  ssems[r]).wait()

        for r in range(nbuf):
            gather(r, r)

        def round_(j, carry):
            for r in range(nbuf):
                i = j * nbuf + r
                g_wait(r)
                store(i, r)

                @pl.when(i + nbuf < nchunks)
                def _():
                    s_wait(r)
                    gather(i + nbuf, r)
            return carry

        lax.fori_loop(0, nchunks // nbuf, round_, 0)
        for r in range(nbuf):
            s_wait(r)

    scratch = ([pltpu.VMEM((nchunks, c), jnp.int32)]
               + [pltpu.VMEM((c, d), jnp.float32) for _ in range(nbuf)]
               + [pltpu.SemaphoreType.DMA for _ in range(2 * nbuf)])
    f = pl.kernel(
        body,
        out_type=jax.ShapeDtypeStruct((n_rows, d), jnp.float32),
        mesh=plsc.VectorSubcoreMesh(**_MESH),
        scratch_types=scratch,
    )
    return f(table, idx3)


def _tc_local(h, item_emb, adj, maskf, a_cat):
    b, l, d = h.shape

    def body(h_ref, it_ref, adj_ref, m_ref, a_ref, hl_ref, si_ref):
        hh = h_ref[0]
        aa = a_ref[...]
        ad = adj_ref[0]
        alpha = jnp.full((l, l), -9e15, dtype=jnp.float32)
        for k in range(4):
            hk = hh * aa[k][None, :]
            ek = lax.dot_general(hk, hh, (((1,), (1,)), ((), ())),
                                 preferred_element_type=jnp.float32)
            ek = jnp.where(ek >= 0, ek, 0.2 * ek)
            alpha = jnp.where(ad == (k + 1), ek, alpha)
        alpha = jax.nn.softmax(alpha, axis=-1)
        hl_ref[0] = jnp.dot(alpha, hh, preferred_element_type=jnp.float32)
        m = m_ref[0, 0]
        it = it_ref[0] * m[:, None]
        si_ref[0, 0] = jnp.sum(it, axis=0) / jnp.sum(m)

    return pl.pallas_call(
        body,
        grid=(b,),
        in_specs=[
            pl.BlockSpec((1, l, d), lambda i: (i, 0, 0)),
            pl.BlockSpec((1, l, d), lambda i: (i, 0, 0)),
            pl.BlockSpec((1, l, l), lambda i: (i, 0, 0)),
            pl.BlockSpec((1, 1, l), lambda i: (i, 0, 0)),
            pl.BlockSpec((4, d), lambda i: (0, 0)),
        ],
        out_specs=[
            pl.BlockSpec((1, l, d), lambda i: (i, 0, 0)),
            pl.BlockSpec((1, 1, d), lambda i: (i, 0, 0)),
        ],
        out_shape=[jax.ShapeDtypeStruct((b, l, d), jnp.float32),
                   jax.ShapeDtypeStruct((b, 1, d), jnp.float32)],
    )(h, item_emb, adj, maskf, a_cat)


def _tc_global(self_v, neigh_v, w, si, sel, selt, w1a, w1b, w2c, w3a, w3b,
               mc, s, resid=None):
    b, m, d = self_v.shape
    nblk = m // mc

    def body(*refs):
        if resid is not None:
            (sf_ref, nb_ref, w_ref, si_ref, sel_ref, selt_ref, w1a_ref,
             w1b_ref, w2_ref, w3a_ref, w3b_ref, res_ref, out_ref) = refs
        else:
            (sf_ref, nb_ref, w_ref, si_ref, sel_ref, selt_ref, w1a_ref,
             w1b_ref, w2_ref, w3a_ref, w3b_ref, out_ref) = refs
            res_ref = None
        nb = nb_ref[0]
        sv = si_ref[0, 0]
        x = nb * sv[None, :]
        t = jnp.dot(x, w1a_ref[...], preferred_element_type=jnp.float32)
        t = t + w_ref[0] * w1b_ref[...]
        t = jnp.where(t >= 0, t, 0.2 * t)
        sc = jnp.dot(t, w2_ref[...], preferred_element_type=jnp.float32)
        e = jnp.exp(sc)
        gs = jnp.dot(sel_ref[...], e, preferred_element_type=jnp.float32)
        den = jnp.dot(selt_ref[...], gs, preferred_element_type=jnp.float32)
        p = e / den
        neigh = jnp.dot(sel_ref[...], p * nb,
                        preferred_element_type=jnp.float32)
        o = jnp.dot(sf_ref[0], w3a_ref[...], preferred_element_type=jnp.float32)
        o = o + jnp.dot(neigh, w3b_ref[...], preferred_element_type=jnp.float32)
        o = jnp.maximum(o, 0.0)
        if res_ref is not None:
            o = o + res_ref[0]
        out_ref[0] = o

    in_specs = [
        pl.BlockSpec((1, mc, d), lambda i, j: (i, j, 0)),
        pl.BlockSpec((1, mc * s, d), lambda i, j: (i, j, 0)),
        pl.BlockSpec((1, mc * s, 1), lambda i, j: (i, j, 0)),
        pl.BlockSpec((1, 1, d), lambda i, j: (i, 0, 0)),
        pl.BlockSpec((mc, mc * s), lambda i, j: (0, 0)),
        pl.BlockSpec((mc * s, mc), lambda i, j: (0, 0)),
        pl.BlockSpec((d, d), lambda i, j: (0, 0)),
        pl.BlockSpec((1, d), lambda i, j: (0, 0)),
        pl.BlockSpec((d, 1), lambda i, j: (0, 0)),
        pl.BlockSpec((d, d), lambda i, j: (0, 0)),
        pl.BlockSpec((d, d), lambda i, j: (0, 0)),
    ]
    args = [self_v, neigh_v, w, si, sel, selt, w1a, w1b, w2c, w3a, w3b]
    if resid is not None:
        in_specs.append(pl.BlockSpec((1, mc, d), lambda i, j: (i, j, 0)))
        args.append(resid)

    return pl.pallas_call(
        body,
        grid=(b, nblk),
        in_specs=in_specs,
        out_specs=pl.BlockSpec((1, mc, d), lambda i, j: (i, j, 0)),
        out_shape=jax.ShapeDtypeStruct((b, m, d), jnp.float32),
        compiler_params=pltpu.CompilerParams(
            dimension_semantics=("parallel", "parallel")),
    )(*args)


def kernel(inputs, adj, mask_item, item, adj_all, num, embedding,
           a0, a1, a2, a3, gw1, gw2, gw3):
    b, l = inputs.shape
    s = adj_all.shape[1]
    d = embedding.shape[1]

    adj_flat = adj_all.reshape(-1, 128)
    num_flat = num.reshape(-1, 128)

    t0 = inputs.reshape(-1)
    t0p = _pad_to(t0, NW * 64).reshape(NW, 64)
    a1_rows, n1_rows = _sc_chase(adj_flat, num_flat, t0p, s)

    n_t1 = b * l * s
    t1 = a1_rows[:b * l, :s].reshape(-1)
    w1n = n1_rows[:b * l, :s].reshape(b, l, s)
    t1p = _pad_to(t1, NW * 640).reshape(NW, 640)
    a2_rows, n2_rows = _sc_chase(adj_flat, num_flat, t1p, s)

    t2 = a2_rows[:n_t1, :s].reshape(-1)
    w2n = n2_rows[:n_t1, :s].reshape(b, l * s, s)
    t2p = t2.reshape(NW, 75, 96)

    hi_idx = jnp.concatenate([t0, item.reshape(-1)])
    hip = _pad_to(hi_idx, NW * 128).reshape(NW, 1, 128)
    ghi = _sc_embed_gather(embedding, hip, nbuf=1)
    h = ghi[:b * l].reshape(b, l, d)
    item_emb = ghi[b * l:2 * b * l].reshape(b, l, d)

    e1 = _sc_embed_gather(embedding, t1.reshape(NW, 5, 120), nbuf=5)
    e1 = e1.reshape(b, l * s, d)

    e2 = _sc_embed_gather(embedding, t2p, nbuf=5)
    e2 = e2.reshape(b, l * s * s, d)

    a_cat = jnp.concatenate([a0, a1, a2, a3], axis=1).T
    maskf = mask_item.astype(jnp.float32).reshape(b, 1, l)
    h_local, si = _tc_local(h, item_emb, adj, maskf, a_cat)

    def wsplit(k):
        return (gw1[k, :d], gw1[k, d:].reshape(1, d), gw2[k],
                gw3[k, :d], gw3[k, d:])

    w1a0, w1b0, w2c0, w3a0, w3b0 = wsplit(0)
    w1a1, w1b1, w2c1, w3a1, w3b1 = wsplit(1)
    eye = jnp.eye(120, dtype=jnp.float32)
    sel120 = jnp.repeat(eye, s, axis=1)
    sel50 = sel120[:l, :l * s]
    wf1 = w1n.reshape(b, l * s, 1)
    wf2 = w2n.reshape(b, l * s * s, 1)

    v0 = _tc_global(h, e1, wf1, si, sel50, sel50.T,
                    w1a0, w1b0, w2c0, w3a0, w3b0, mc=l, s=s)
    v1 = _tc_global(e1, e2, wf2, si, sel120, sel120.T,
                    w1a0, w1b0, w2c0, w3a0, w3b0, mc=120, s=s)
    out = _tc_global(v0, v1, wf1, si, sel50, sel50.T,
                     w1a1, w1b1, w2c1, w3a1, w3b1, mc=l, s=s,
                     resid=h_local)
    return out


# speedup vs baseline: 2.4865x; 1.2568x over previous
import functools

import jax
import jax.numpy as jnp
from jax import lax
from jax.experimental import pallas as pl
from jax.experimental.pallas import tpu as pltpu
from jax.experimental.pallas import tpu_sc as plsc

DIM = 128
NC, NS = 2, 16
NW = NC * NS

_MESH = dict(core_axis_name="c", subcore_axis_name="s")


def _wid():
    return lax.axis_index("s") * NC + lax.axis_index("c")


def _pad_to(x, n):
    return jnp.concatenate([x, jnp.arange(n - x.shape[0], dtype=x.dtype)])


def _sc_chase(adj_flat, num_flat, idx2, s):
    nw, npt = idx2.shape
    assert npt % 64 == 0
    ndma = npt // 64
    ngrp = npt // 16
    ngran = adj_flat.shape[0]
    n_rows = nw * npt

    def body(adj_hbm, num_hbm, idx_hbm, oa_hbm, on_hbm,
             idx_v, bidx_v, ablk, nblk, oa_v, on_v, sema, semn):
        wid = _wid()
        pltpu.sync_copy(idx_hbm.at[wid], idx_v)
        lane = lax.iota(jnp.int32, 16)

        def issue(i, carry):
            for q in range(4):
                t16 = idx_v[pl.ds(i * 64 + q * 16, 16)]
                w = t16 * s
                g0 = lax.shift_right_logical(w, 4)
                bidx_v[pl.ds(i * 128 + 32 * q, 16)] = g0
                bidx_v[pl.ds(i * 128 + 32 * q + 16, 16)] = (
                    jnp.minimum(g0 + 1, ngran - 1))
            pltpu.async_copy(adj_hbm.at[bidx_v.at[pl.ds(i * 128, 128)]],
                             ablk.at[pl.ds(i * 128, 128)], sema)
            pltpu.async_copy(num_hbm.at[bidx_v.at[pl.ds(i * 128, 128)]],
                             nblk.at[pl.ds(i * 128, 128)], semn)
            return carry

        lax.fori_loop(0, ndma, issue, 0)

        def drain(i, carry):
            pltpu.make_async_copy(adj_hbm.at[bidx_v.at[pl.ds(0, 128)]],
                                  ablk.at[pl.ds(0, 128)], sema).wait()
            pltpu.make_async_copy(num_hbm.at[bidx_v.at[pl.ds(0, 128)]],
                                  nblk.at[pl.ds(0, 128)], semn).wait()
            return carry

        lax.fori_loop(0, ndma, drain, 0)

        def extract(j, carry):
            t16 = idx_v[pl.ds(j * 16, 16)]
            o = lax.bitwise_and(t16 * s, 15)
            for k in range(s):
                ck = o + k
                row = 32 * j + lane + 16 * lax.shift_right_logical(ck, 4)
                col = lax.bitwise_and(ck, 15)
                va = plsc.load_gather(ablk, [row, col])
                vn = plsc.load_gather(nblk, [row, col])
                orow = 16 * j + lane
                kv = jnp.full((16,), k, jnp.int32)
                plsc.store_scatter(oa_v, [orow, kv], va)
                plsc.store_scatter(on_v, [orow, kv], vn)
            return carry

        lax.fori_loop(0, ngrp, extract, 0)
        pltpu.sync_copy(oa_v, oa_hbm.at[pl.ds(wid * npt, npt)])
        pltpu.sync_copy(on_v, on_hbm.at[pl.ds(wid * npt, npt)])

    f = pl.kernel(
        body,
        out_type=(jax.ShapeDtypeStruct((n_rows, 16), jnp.int32),
                  jax.ShapeDtypeStruct((n_rows, 16), jnp.float32)),
        mesh=plsc.VectorSubcoreMesh(**_MESH),
        scratch_types=[pltpu.VMEM((npt,), jnp.int32),
                       pltpu.VMEM((2 * npt,), jnp.int32),
                       pltpu.VMEM((2 * npt, 16), jnp.int32),
                       pltpu.VMEM((2 * npt, 16), jnp.float32),
                       pltpu.VMEM((npt, 16), jnp.int32),
                       pltpu.VMEM((npt, 16), jnp.float32),
                       pltpu.SemaphoreType.DMA,
                       pltpu.SemaphoreType.DMA],
        compiler_params=pltpu.CompilerParams(
            needs_layout_passes=False, use_tc_tiling_on_sc=False),
    )
    return f(adj_flat, num_flat, idx2)


def _sc_embed_gather(table, idx3, nbuf):
    nw, nchunks, c = idx3.shape
    d = table.shape[1]
    n_rows = nw * nchunks * c
    assert nchunks % nbuf == 0

    def body(tab_hbm, idx_hbm, out_hbm, idx_v, *rest):
        bufs = rest[:nbuf]
        gsems = rest[nbuf:2 * nbuf]
        ssems = rest[2 * nbuf:3 * nbuf]
        wid = _wid()
        pltpu.sync_copy(idx_hbm.at[wid], idx_v)

        def gather(i, r):
            pltpu.async_copy(tab_hbm.at[idx_v.at[i]], bufs[r], gsems[r])

        def store(i, r):
            pltpu.async_copy(
                bufs[r], out_hbm.at[pl.ds((wid * nchunks + i) * c, c)],
                ssems[r])

        def g_wait(r):
            pltpu.make_async_copy(tab_hbm.at[idx_v.at[0]], bufs[r],
                                  gsems[r]).wait()

        def s_wait(r):
            pltpu.make_async_copy(bufs[r], out_hbm.at[pl.ds(0, c)],
                                  ssems[r]).wait()

        for r in range(nbuf):
            gather(r, r)

        def round_(j, carry):
            for r in range(nbuf):
                i = j * nbuf + r
                g_wait(r)
                store(i, r)

                @pl.when(i + nbuf < nchunks)
                def _():
                    s_wait(r)
                    gather(i + nbuf, r)
            return carry

        lax.fori_loop(0, nchunks // nbuf, round_, 0)
        for r in range(nbuf):
            s_wait(r)

    scratch = ([pltpu.VMEM((nchunks, c), jnp.int32)]
               + [pltpu.VMEM((c, d), jnp.float32) for _ in range(nbuf)]
               + [pltpu.SemaphoreType.DMA for _ in range(2 * nbuf)])
    f = pl.kernel(
        body,
        out_type=jax.ShapeDtypeStruct((n_rows, d), jnp.float32),
        mesh=plsc.VectorSubcoreMesh(**_MESH),
        scratch_types=scratch,
    )
    return f(table, idx3)


def _tc_local(h, item_emb, adj, maskf, a_cat):
    b, l, d = h.shape

    def body(h_ref, it_ref, adj_ref, m_ref, a_ref, hl_ref, si_ref):
        hh = h_ref[0]
        aa = a_ref[...]
        ad = adj_ref[0]
        alpha = jnp.full((l, l), -9e15, dtype=jnp.float32)
        for k in range(4):
            hk = hh * aa[k][None, :]
            ek = lax.dot_general(hk, hh, (((1,), (1,)), ((), ())),
                                 preferred_element_type=jnp.float32)
            ek = jnp.where(ek >= 0, ek, 0.2 * ek)
            alpha = jnp.where(ad == (k + 1), ek, alpha)
        alpha = jax.nn.softmax(alpha, axis=-1)
        hl_ref[0] = jnp.dot(alpha, hh, preferred_element_type=jnp.float32)
        m = m_ref[0, 0]
        it = it_ref[0] * m[:, None]
        si_ref[0, 0] = jnp.sum(it, axis=0) / jnp.sum(m)

    return pl.pallas_call(
        body,
        grid=(b,),
        in_specs=[
            pl.BlockSpec((1, l, d), lambda i: (i, 0, 0)),
            pl.BlockSpec((1, l, d), lambda i: (i, 0, 0)),
            pl.BlockSpec((1, l, l), lambda i: (i, 0, 0)),
            pl.BlockSpec((1, 1, l), lambda i: (i, 0, 0)),
            pl.BlockSpec((4, d), lambda i: (0, 0)),
        ],
        out_specs=[
            pl.BlockSpec((1, l, d), lambda i: (i, 0, 0)),
            pl.BlockSpec((1, 1, d), lambda i: (i, 0, 0)),
        ],
        out_shape=[jax.ShapeDtypeStruct((b, l, d), jnp.float32),
                   jax.ShapeDtypeStruct((b, 1, d), jnp.float32)],
    )(h, item_emb, adj, maskf, a_cat)


def _tc_global(self_v, neigh_v, w, si, sel, w1a, w1b, w2c, w3a, w3b,
               mc, s, resid=None):
    b, m, d = self_v.shape
    nblk = m // mc

    def body(*refs):
        if resid is not None:
            (sf_ref, nb_ref, w_ref, si_ref, sel_ref, w1a_ref,
             w1b_ref, w2_ref, w3a_ref, w3b_ref, res_ref, out_ref) = refs
        else:
            (sf_ref, nb_ref, w_ref, si_ref, sel_ref, w1a_ref,
             w1b_ref, w2_ref, w3a_ref, w3b_ref, out_ref) = refs
            res_ref = None
        nb = nb_ref[0]
        sv = si_ref[0, 0]
        w1s = sv[:, None] * w1a_ref[...]
        t = jnp.dot(nb, w1s, preferred_element_type=jnp.float32)
        t = t + w_ref[0] * w1b_ref[...]
        t = jnp.where(t >= 0, t, 0.2 * t)
        sc = jnp.dot(t, w2_ref[...], preferred_element_type=jnp.float32)
        e = jnp.exp(sc)
        gs = jnp.dot(sel_ref[...], e, preferred_element_type=jnp.float32)
        num = jnp.dot(sel_ref[...], e * nb,
                      preferred_element_type=jnp.float32)
        neigh = num / gs
        o = jnp.dot(sf_ref[0], w3a_ref[...], preferred_element_type=jnp.float32)
        o = o + jnp.dot(neigh, w3b_ref[...], preferred_element_type=jnp.float32)
        o = jnp.maximum(o, 0.0)
        if res_ref is not None:
            o = o + res_ref[0]
        out_ref[0] = o

    in_specs = [
        pl.BlockSpec((1, mc, d), lambda i, j: (i, j, 0)),
        pl.BlockSpec((1, mc * s, d), lambda i, j: (i, j, 0)),
        pl.BlockSpec((1, mc * s, 1), lambda i, j: (i, j, 0)),
        pl.BlockSpec((1, 1, d), lambda i, j: (i, 0, 0)),
        pl.BlockSpec((mc, mc * s), lambda i, j: (0, 0)),
        pl.BlockSpec((d, d), lambda i, j: (0, 0)),
        pl.BlockSpec((1, d), lambda i, j: (0, 0)),
        pl.BlockSpec((d, 1), lambda i, j: (0, 0)),
        pl.BlockSpec((d, d), lambda i, j: (0, 0)),
        pl.BlockSpec((d, d), lambda i, j: (0, 0)),
    ]
    args = [self_v, neigh_v, w, si, sel, w1a, w1b, w2c, w3a, w3b]
    if resid is not None:
        in_specs.append(pl.BlockSpec((1, mc, d), lambda i, j: (i, j, 0)))
        args.append(resid)

    return pl.pallas_call(
        body,
        grid=(b, nblk),
        in_specs=in_specs,
        out_specs=pl.BlockSpec((1, mc, d), lambda i, j: (i, j, 0)),
        out_shape=jax.ShapeDtypeStruct((b, m, d), jnp.float32),
        compiler_params=pltpu.CompilerParams(
            dimension_semantics=("parallel", "parallel")),
    )(*args)


def kernel(inputs, adj, mask_item, item, adj_all, num, embedding,
           a0, a1, a2, a3, gw1, gw2, gw3):
    b, l = inputs.shape
    s = adj_all.shape[1]
    d = embedding.shape[1]

    adj_flat = adj_all.reshape(-1, 16)
    num_flat = num.reshape(-1, 16)

    t0 = inputs.reshape(-1)
    t0p = _pad_to(t0, NW * 64).reshape(NW, 64)
    a1_rows, n1_rows = _sc_chase(adj_flat, num_flat, t0p, s)

    n_t1 = b * l * s
    t1 = a1_rows[:b * l, :s].reshape(-1)
    w1n = n1_rows[:b * l, :s].reshape(b, l, s)
    t1p = _pad_to(t1, NW * 640).reshape(NW, 640)
    a2_rows, n2_rows = _sc_chase(adj_flat, num_flat, t1p, s)

    t2 = a2_rows[:n_t1, :s].reshape(-1)
    w2n = n2_rows[:n_t1, :s].reshape(b, l * s, s)
    t2p = t2.reshape(NW, 75, 96)

    hi_idx = jnp.concatenate([t0, item.reshape(-1)])
    hip = _pad_to(hi_idx, NW * 128).reshape(NW, 1, 128)
    ghi = _sc_embed_gather(embedding, hip, nbuf=1)
    h = ghi[:b * l].reshape(b, l, d)
    item_emb = ghi[b * l:2 * b * l].reshape(b, l, d)

    e1 = _sc_embed_gather(embedding, t1.reshape(NW, 5, 120), nbuf=5)
    e1 = e1.reshape(b, l * s, d)

    e2 = _sc_embed_gather(embedding, t2p, nbuf=5)
    e2 = e2.reshape(b, l * s * s, d)

    a_cat = jnp.concatenate([a0, a1, a2, a3], axis=1).T
    maskf = mask_item.astype(jnp.float32).reshape(b, 1, l)
    h_local, si = _tc_local(h, item_emb, adj, maskf, a_cat)

    def wsplit(k):
        return (gw1[k, :d], gw1[k, d:].reshape(1, d), gw2[k],
                gw3[k, :d], gw3[k, d:])

    w1a0, w1b0, w2c0, w3a0, w3b0 = wsplit(0)
    w1a1, w1b1, w2c1, w3a1, w3b1 = wsplit(1)
    eye = jnp.eye(120, dtype=jnp.float32)
    sel120 = jnp.repeat(eye, s, axis=1)
    sel50 = sel120[:l, :l * s]
    wf1 = w1n.reshape(b, l * s, 1)
    wf2 = w2n.reshape(b, l * s * s, 1)

    v0 = _tc_global(h, e1, wf1, si, sel50,
                    w1a0, w1b0, w2c0, w3a0, w3b0, mc=l, s=s)
    v1 = _tc_global(e1, e2, wf2, si, sel120,
                    w1a0, w1b0, w2c0, w3a0, w3b0, mc=120, s=s)
    out = _tc_global(v0, v1, wf1, si, sel50,
                     w1a1, w1b1, w2c1, w3a1, w3b1, mc=l, s=s,
                     resid=h_local)
    return out


# speedup vs baseline: 2.5085x; 1.0089x over previous
import functools

import jax
import jax.numpy as jnp
from jax import lax
from jax.experimental import pallas as pl
from jax.experimental.pallas import tpu as pltpu
from jax.experimental.pallas import tpu_sc as plsc

DIM = 128
NC, NS = 2, 16
NW = NC * NS

_MESH = dict(core_axis_name="c", subcore_axis_name="s")


def _wid():
    return lax.axis_index("s") * NC + lax.axis_index("c")


def _pad_to(x, n):
    return jnp.concatenate([x, jnp.arange(n - x.shape[0], dtype=x.dtype)])


def _sc_chase(table16, idx2, s):
    nw, npt = idx2.shape
    assert npt % 64 == 0
    ndma = npt // 64
    ngrp = npt // 16
    ngran = table16.shape[0]
    n_rows = nw * npt

    def body(tab_hbm, idx_hbm, out_hbm, idx_v, bidx_v, blk, out_v, sem):
        wid = _wid()
        pltpu.sync_copy(idx_hbm.at[wid], idx_v)
        lane = lax.iota(jnp.int32, 16)

        def issue(i, carry):
            for q in range(4):
                t16 = idx_v[pl.ds(i * 64 + q * 16, 16)]
                w = t16 * s
                g0 = lax.shift_right_logical(w, 4)
                bidx_v[pl.ds(i * 128 + 32 * q, 16)] = g0
                bidx_v[pl.ds(i * 128 + 32 * q + 16, 16)] = (
                    jnp.minimum(g0 + 1, ngran - 1))
            pltpu.async_copy(tab_hbm.at[bidx_v.at[pl.ds(i * 128, 128)]],
                             blk.at[pl.ds(i * 128, 128)], sem)
            return carry

        lax.fori_loop(0, ndma, issue, 0)

        def drain(i, carry):
            pltpu.make_async_copy(tab_hbm.at[bidx_v.at[pl.ds(0, 128)]],
                                  blk.at[pl.ds(0, 128)], sem).wait()
            return carry

        lax.fori_loop(0, ndma, drain, 0)

        def extract(j, carry):
            t16 = idx_v[pl.ds(j * 16, 16)]
            o = lax.bitwise_and(t16 * s, 15)
            orow = 16 * j + lane
            for k in range(s):
                ck = o + k
                row = 32 * j + lane + 16 * lax.shift_right_logical(ck, 4)
                col = lax.bitwise_and(ck, 15)
                v = plsc.load_gather(blk, [row, col])
                kv = jnp.full((16,), k, jnp.int32)
                plsc.store_scatter(out_v, [orow, kv], v)
            return carry

        lax.fori_loop(0, ngrp, extract, 0)
        pltpu.sync_copy(out_v, out_hbm.at[pl.ds(wid * npt, npt)])

    f = pl.kernel(
        body,
        out_type=jax.ShapeDtypeStruct((n_rows, 16), table16.dtype),
        mesh=plsc.VectorSubcoreMesh(**_MESH),
        scratch_types=[pltpu.VMEM((npt,), jnp.int32),
                       pltpu.VMEM((2 * npt,), jnp.int32),
                       pltpu.VMEM((2 * npt, 16), table16.dtype),
                       pltpu.VMEM((npt, 16), table16.dtype),
                       pltpu.SemaphoreType.DMA],
        compiler_params=pltpu.CompilerParams(
            needs_layout_passes=False, use_tc_tiling_on_sc=False),
    )
    return f(table16, idx2)


def _sc_embed_gather(table, idx3, nbuf):
    nw, nchunks, c = idx3.shape
    d = table.shape[1]
    n_rows = nw * nchunks * c
    assert nchunks % nbuf == 0

    def body(tab_hbm, idx_hbm, out_hbm, idx_v, *rest):
        bufs = rest[:nbuf]
        gsems = rest[nbuf:2 * nbuf]
        ssems = rest[2 * nbuf:3 * nbuf]
        wid = _wid()
        pltpu.sync_copy(idx_hbm.at[wid], idx_v)

        def gather(i, r):
            pltpu.async_copy(tab_hbm.at[idx_v.at[i]], bufs[r], gsems[r])

        def store(i, r):
            pltpu.async_copy(
                bufs[r], out_hbm.at[pl.ds((wid * nchunks + i) * c, c)],
                ssems[r])

        def g_wait(r):
            pltpu.make_async_copy(tab_hbm.at[idx_v.at[0]], bufs[r],
                                  gsems[r]).wait()

        def s_wait(r):
            pltpu.make_async_copy(bufs[r], out_hbm.at[pl.ds(0, c)],
                                  ssems[r]).wait()

        for r in range(nbuf):
            gather(r, r)

        def round_(j, carry):
            for r in range(nbuf):
                i = j * nbuf + r
                g_wait(r)
                store(i, r)

                @pl.when(i + nbuf < nchunks)
                def _():
                    s_wait(r)
                    gather(i + nbuf, r)
            return carry

        lax.fori_loop(0, nchunks // nbuf, round_, 0)
        for r in range(nbuf):
            s_wait(r)

    scratch = ([pltpu.VMEM((nchunks, c), jnp.int32)]
               + [pltpu.VMEM((c, d), jnp.float32) for _ in range(nbuf)]
               + [pltpu.SemaphoreType.DMA for _ in range(2 * nbuf)])
    f = pl.kernel(
        body,
        out_type=jax.ShapeDtypeStruct((n_rows, d), jnp.float32),
        mesh=plsc.VectorSubcoreMesh(**_MESH),
        scratch_types=scratch,
    )
    return f(table, idx3)


def _tc_local(h, item_emb, adj, maskf, a_cat):
    b, l, d = h.shape

    def body(h_ref, it_ref, adj_ref, m_ref, a_ref, hl_ref, si_ref):
        hh = h_ref[0]
        aa = a_ref[...]
        ad = adj_ref[0]
        alpha = jnp.full((l, l), -9e15, dtype=jnp.float32)
        for k in range(4):
            hk = hh * aa[k][None, :]
            ek = lax.dot_general(hk, hh, (((1,), (1,)), ((), ())),
                                 preferred_element_type=jnp.float32)
            ek = jnp.where(ek >= 0, ek, 0.2 * ek)
            alpha = jnp.where(ad == (k + 1), ek, alpha)
        alpha = jax.nn.softmax(alpha, axis=-1)
        hl_ref[0] = jnp.dot(alpha, hh, preferred_element_type=jnp.float32)
        m = m_ref[0, 0]
        it = it_ref[0] * m[:, None]
        si_ref[0, 0] = jnp.sum(it, axis=0) / jnp.sum(m)

    return pl.pallas_call(
        body,
        grid=(b,),
        in_specs=[
            pl.BlockSpec((1, l, d), lambda i: (i, 0, 0)),
            pl.BlockSpec((1, l, d), lambda i: (i, 0, 0)),
            pl.BlockSpec((1, l, l), lambda i: (i, 0, 0)),
            pl.BlockSpec((1, 1, l), lambda i: (i, 0, 0)),
            pl.BlockSpec((4, d), lambda i: (0, 0)),
        ],
        out_specs=[
            pl.BlockSpec((1, l, d), lambda i: (i, 0, 0)),
            pl.BlockSpec((1, 1, d), lambda i: (i, 0, 0)),
        ],
        out_shape=[jax.ShapeDtypeStruct((b, l, d), jnp.float32),
                   jax.ShapeDtypeStruct((b, 1, d), jnp.float32)],
    )(h, item_emb, adj, maskf, a_cat)


def _tc_global(self_v, neigh_v, w, si, sel, w1a, w1b, w2c, w3a, w3b,
               mc, s, resid=None):
    b, m, d = self_v.shape
    nblk = m // mc

    def body(*refs):
        if resid is not None:
            (sf_ref, nb_ref, w_ref, si_ref, sel_ref, w1a_ref,
             w1b_ref, w2_ref, w3a_ref, w3b_ref, res_ref, out_ref) = refs
        else:
            (sf_ref, nb_ref, w_ref, si_ref, sel_ref, w1a_ref,
             w1b_ref, w2_ref, w3a_ref, w3b_ref, out_ref) = refs
            res_ref = None
        nb = nb_ref[0]
        sv = si_ref[0, 0]
        w1s = sv[:, None] * w1a_ref[...]
        t = jnp.dot(nb, w1s, preferred_element_type=jnp.float32)
        t = t + w_ref[0] * w1b_ref[...]
        t = jnp.where(t >= 0, t, 0.2 * t)
        sc = jnp.dot(t, w2_ref[...], preferred_element_type=jnp.float32)
        e = jnp.exp(sc)
        gs = jnp.dot(sel_ref[...], e, preferred_element_type=jnp.float32)
        num = jnp.dot(sel_ref[...], e * nb,
                      preferred_element_type=jnp.float32)
        neigh = num / gs
        o = jnp.dot(sf_ref[0], w3a_ref[...], preferred_element_type=jnp.float32)
        o = o + jnp.dot(neigh, w3b_ref[...], preferred_element_type=jnp.float32)
        o = jnp.maximum(o, 0.0)
        if res_ref is not None:
            o = o + res_ref[0]
        out_ref[0] = o

    in_specs = [
        pl.BlockSpec((1, mc, d), lambda i, j: (i, j, 0)),
        pl.BlockSpec((1, mc * s, d), lambda i, j: (i, j, 0)),
        pl.BlockSpec((1, mc * s, 1), lambda i, j: (i, j, 0)),
        pl.BlockSpec((1, 1, d), lambda i, j: (i, 0, 0)),
        pl.BlockSpec((mc, mc * s), lambda i, j: (0, 0)),
        pl.BlockSpec((d, d), lambda i, j: (0, 0)),
        pl.BlockSpec((1, d), lambda i, j: (0, 0)),
        pl.BlockSpec((d, 1), lambda i, j: (0, 0)),
        pl.BlockSpec((d, d), lambda i, j: (0, 0)),
        pl.BlockSpec((d, d), lambda i, j: (0, 0)),
    ]
    args = [self_v, neigh_v, w, si, sel, w1a, w1b, w2c, w3a, w3b]
    if resid is not None:
        in_specs.append(pl.BlockSpec((1, mc, d), lambda i, j: (i, j, 0)))
        args.append(resid)

    return pl.pallas_call(
        body,
        grid=(b, nblk),
        in_specs=in_specs,
        out_specs=pl.BlockSpec((1, mc, d), lambda i, j: (i, j, 0)),
        out_shape=jax.ShapeDtypeStruct((b, m, d), jnp.float32),
        compiler_params=pltpu.CompilerParams(
            dimension_semantics=("parallel", "parallel")),
    )(*args)


def kernel(inputs, adj, mask_item, item, adj_all, num, embedding,
           a0, a1, a2, a3, gw1, gw2, gw3):
    b, l = inputs.shape
    s = adj_all.shape[1]
    d = embedding.shape[1]

    adj16 = adj_all.reshape(-1, 16)
    num16 = num.reshape(-1, 16)

    t0 = inputs.reshape(-1)
    t0p = _pad_to(t0, NW * 64).reshape(NW, 64)
    a1_rows = _sc_chase(adj16, t0p, s)

    n_t1 = b * l * s
    t1 = a1_rows[:b * l, :s].reshape(-1)
    t1p = _pad_to(t1, NW * 640).reshape(NW, 640)
    a2_rows = _sc_chase(adj16, t1p, s)

    nall = _sc_chase(num16, jnp.concatenate(
        [t0p.reshape(-1), t1p.reshape(-1)]).reshape(NW, 704), s)
    w1n = nall[:b * l, :s].reshape(b, l, s)
    w2n = nall[NW * 64:NW * 64 + n_t1, :s].reshape(b, l * s, s)

    t2 = a2_rows[:n_t1, :s].reshape(-1)
    t2p = t2.reshape(NW, 75, 96)

    hi_idx = jnp.concatenate([t0, item.reshape(-1)])
    hip = _pad_to(hi_idx, NW * 128).reshape(NW, 1, 128)
    ghi = _sc_embed_gather(embedding, hip, nbuf=1)
    h = ghi[:b * l].reshape(b, l, d)
    item_emb = ghi[b * l:2 * b * l].reshape(b, l, d)

    e1 = _sc_embed_gather(embedding, t1.reshape(NW, 5, 120), nbuf=5)
    e1 = e1.reshape(b, l * s, d)

    e2 = _sc_embed_gather(embedding, t2p, nbuf=5)
    e2 = e2.reshape(b, l * s * s, d)

    a_cat = jnp.concatenate([a0, a1, a2, a3], axis=1).T
    maskf = mask_item.astype(jnp.float32).reshape(b, 1, l)
    h_local, si = _tc_local(h, item_emb, adj, maskf, a_cat)

    def wsplit(k):
        return (gw1[k, :d], gw1[k, d:].reshape(1, d), gw2[k],
                gw3[k, :d], gw3[k, d:])

    w1a0, w1b0, w2c0, w3a0, w3b0 = wsplit(0)
    w1a1, w1b1, w2c1, w3a1, w3b1 = wsplit(1)
    eye = jnp.eye(120, dtype=jnp.float32)
    sel120 = jnp.repeat(eye, s, axis=1)
    sel50 = sel120[:l, :l * s]
    wf1 = w1n.reshape(b, l * s, 1)
    wf2 = w2n.reshape(b, l * s * s, 1)

    v0 = _tc_global(h, e1, wf1, si, sel50,
                    w1a0, w1b0, w2c0, w3a0, w3b0, mc=l, s=s)
    v1 = _tc_global(e1, e2, wf2, si, sel120,
                    w1a0, w1b0, w2c0, w3a0, w3b0, mc=120, s=s)
    out = _tc_global(v0, v1, wf1, si, sel50,
                     w1a1, w1b1, w2c1, w3a1, w3b1, mc=l, s=s,
                     resid=h_local)
    return out


# speedup vs baseline: 2.6582x; 1.0597x over previous
import functools

import jax
import jax.numpy as jnp
from jax import lax
from jax.experimental import pallas as pl
from jax.experimental.pallas import tpu as pltpu
from jax.experimental.pallas import tpu_sc as plsc

DIM = 128
NC, NS = 2, 16
NW = NC * NS

_MESH = dict(core_axis_name="c", subcore_axis_name="s")


def _wid():
    return lax.axis_index("s") * NC + lax.axis_index("c")


def _pad_to(x, n):
    return jnp.concatenate([x, jnp.arange(n - x.shape[0], dtype=x.dtype)])


def _sc_chase(table16, idx2, s):
    nw, npt = idx2.shape
    assert npt % 64 == 0
    ndma = npt // 64
    ngrp = npt // 16
    ngran = table16.shape[0]
    n_rows = nw * npt

    def body(tab_hbm, idx_hbm, out_hbm, idx_v, bidx_v, blk, out_v, sem):
        wid = _wid()
        pltpu.sync_copy(idx_hbm.at[wid], idx_v)
        lane = lax.iota(jnp.int32, 16)

        def issue(i, carry):
            for q in range(4):
                t16 = idx_v[pl.ds(i * 64 + q * 16, 16)]
                w = t16 * s
                g0 = lax.shift_right_logical(w, 4)
                bidx_v[pl.ds(i * 128 + 32 * q, 16)] = g0
                bidx_v[pl.ds(i * 128 + 32 * q + 16, 16)] = (
                    jnp.minimum(g0 + 1, ngran - 1))
            pltpu.async_copy(tab_hbm.at[bidx_v.at[pl.ds(i * 128, 128)]],
                             blk.at[pl.ds(i * 128, 128)], sem)
            return carry

        lax.fori_loop(0, ndma, issue, 0)

        def drain(i, carry):
            pltpu.make_async_copy(tab_hbm.at[bidx_v.at[pl.ds(0, 128)]],
                                  blk.at[pl.ds(0, 128)], sem).wait()
            return carry

        lax.fori_loop(0, ndma, drain, 0)

        def extract(j, carry):
            t16 = idx_v[pl.ds(j * 16, 16)]
            o = lax.bitwise_and(t16 * s, 15)
            orow = 16 * j + lane
            for k in range(s):
                ck = o + k
                row = 32 * j + lane + 16 * lax.shift_right_logical(ck, 4)
                col = lax.bitwise_and(ck, 15)
                v = plsc.load_gather(blk, [row, col])
                kv = jnp.full((16,), k, jnp.int32)
                plsc.store_scatter(out_v, [orow, kv], v)
            return carry

        lax.fori_loop(0, ngrp, extract, 0)
        pltpu.sync_copy(out_v, out_hbm.at[pl.ds(wid * npt, npt)])

    f = pl.kernel(
        body,
        out_type=jax.ShapeDtypeStruct((n_rows, 16), table16.dtype),
        mesh=plsc.VectorSubcoreMesh(**_MESH),
        scratch_types=[pltpu.VMEM((npt,), jnp.int32),
                       pltpu.VMEM((2 * npt,), jnp.int32),
                       pltpu.VMEM((2 * npt, 16), table16.dtype),
                       pltpu.VMEM((npt, 16), table16.dtype),
                       pltpu.SemaphoreType.DMA],
        compiler_params=pltpu.CompilerParams(
            needs_layout_passes=False, use_tc_tiling_on_sc=False),
    )
    return f(table16, idx2)


def _sc_embed_gather(table, idx3, nbuf):
    nw, nchunks, c = idx3.shape
    d = table.shape[1]
    n_rows = nw * nchunks * c
    assert nchunks % nbuf == 0

    def body(tab_hbm, idx_hbm, out_hbm, idx_v, *rest):
        bufs = rest[:nbuf]
        gsems = rest[nbuf:2 * nbuf]
        ssems = rest[2 * nbuf:3 * nbuf]
        wid = _wid()
        pltpu.sync_copy(idx_hbm.at[wid], idx_v)

        def gather(i, r):
            pltpu.async_copy(tab_hbm.at[idx_v.at[i]], bufs[r], gsems[r])

        def store(i, r):
            pltpu.async_copy(
                bufs[r], out_hbm.at[pl.ds((wid * nchunks + i) * c, c)],
                ssems[r])

        def g_wait(r):
            pltpu.make_async_copy(tab_hbm.at[idx_v.at[0]], bufs[r],
                                  gsems[r]).wait()

        def s_wait(r):
            pltpu.make_async_copy(bufs[r], out_hbm.at[pl.ds(0, c)],
                                  ssems[r]).wait()

        for r in range(nbuf):
            gather(r, r)

        def round_(j, carry):
            for r in range(nbuf):
                i = j * nbuf + r
                g_wait(r)
                store(i, r)

                @pl.when(i + nbuf < nchunks)
                def _():
                    s_wait(r)
                    gather(i + nbuf, r)
            return carry

        lax.fori_loop(0, nchunks // nbuf, round_, 0)
        for r in range(nbuf):
            s_wait(r)

    scratch = ([pltpu.VMEM((nchunks, c), jnp.int32)]
               + [pltpu.VMEM((c, d), jnp.float32) for _ in range(nbuf)]
               + [pltpu.SemaphoreType.DMA for _ in range(2 * nbuf)])
    f = pl.kernel(
        body,
        out_type=jax.ShapeDtypeStruct((n_rows, d), jnp.float32),
        mesh=plsc.VectorSubcoreMesh(**_MESH),
        scratch_types=scratch,
    )
    return f(table, idx3)


def _tc_local(h, item_emb, adj, maskf, a_cat):
    b, l, d = h.shape

    def body(h_ref, it_ref, adj_ref, m_ref, a_ref, hl_ref, si_ref):
        hh = h_ref[0]
        aa = a_ref[...]
        ad = adj_ref[0]
        alpha = jnp.full((l, l), -9e15, dtype=jnp.float32)
        for k in range(4):
            hk = hh * aa[k][None, :]
            ek = lax.dot_general(hk, hh, (((1,), (1,)), ((), ())),
                                 preferred_element_type=jnp.float32)
            ek = jnp.where(ek >= 0, ek, 0.2 * ek)
            alpha = jnp.where(ad == (k + 1), ek, alpha)
        alpha = jax.nn.softmax(alpha, axis=-1)
        hl_ref[0] = jnp.dot(alpha, hh, preferred_element_type=jnp.float32)
        m = m_ref[0, 0]
        it = it_ref[0] * m[:, None]
        si_ref[0, 0] = jnp.sum(it, axis=0) / jnp.sum(m)

    return pl.pallas_call(
        body,
        grid=(b,),
        in_specs=[
            pl.BlockSpec((1, l, d), lambda i: (i, 0, 0)),
            pl.BlockSpec((1, l, d), lambda i: (i, 0, 0)),
            pl.BlockSpec((1, l, l), lambda i: (i, 0, 0)),
            pl.BlockSpec((1, 1, l), lambda i: (i, 0, 0)),
            pl.BlockSpec((4, d), lambda i: (0, 0)),
        ],
        out_specs=[
            pl.BlockSpec((1, l, d), lambda i: (i, 0, 0)),
            pl.BlockSpec((1, 1, d), lambda i: (i, 0, 0)),
        ],
        out_shape=[jax.ShapeDtypeStruct((b, l, d), jnp.float32),
                   jax.ShapeDtypeStruct((b, 1, d), jnp.float32)],
    )(h, item_emb, adj, maskf, a_cat)


def _tc_global(self_v, neigh_v, w, si, sel, w1a, w1b, w2c, w3a, w3b,
               mc, s, resid=None):
    b, m, d = self_v.shape
    nblk = m // mc

    def body(*refs):
        if resid is not None:
            (sf_ref, nb_ref, w_ref, si_ref, sel_ref, w1a_ref,
             w1b_ref, w2_ref, w3a_ref, w3b_ref, res_ref, out_ref) = refs
        else:
            (sf_ref, nb_ref, w_ref, si_ref, sel_ref, w1a_ref,
             w1b_ref, w2_ref, w3a_ref, w3b_ref, out_ref) = refs
            res_ref = None
        nb = nb_ref[0]
        sv = si_ref[0, 0]
        w1s = sv[:, None] * w1a_ref[...]
        t = jnp.dot(nb, w1s, preferred_element_type=jnp.float32)
        t = t + w_ref[0] * w1b_ref[...]
        t = jnp.where(t >= 0, t, 0.2 * t)
        sc = jnp.dot(t, w2_ref[...], preferred_element_type=jnp.float32)
        e = jnp.exp(sc)
        gs = jnp.dot(sel_ref[...], e, preferred_element_type=jnp.float32)
        num = jnp.dot(sel_ref[...], e * nb,
                      preferred_element_type=jnp.float32)
        neigh = num / gs
        o = jnp.dot(sf_ref[0], w3a_ref[...], preferred_element_type=jnp.float32)
        o = o + jnp.dot(neigh, w3b_ref[...], preferred_element_type=jnp.float32)
        o = jnp.maximum(o, 0.0)
        if res_ref is not None:
            o = o + res_ref[0]
        out_ref[0] = o

    in_specs = [
        pl.BlockSpec((1, mc, d), lambda i, j: (i, j, 0)),
        pl.BlockSpec((1, mc * s, d), lambda i, j: (i, j, 0)),
        pl.BlockSpec((1, mc * s, 1), lambda i, j: (i, j, 0)),
        pl.BlockSpec((1, 1, d), lambda i, j: (i, 0, 0)),
        pl.BlockSpec((mc, mc * s), lambda i, j: (0, 0)),
        pl.BlockSpec((d, d), lambda i, j: (0, 0)),
        pl.BlockSpec((1, d), lambda i, j: (0, 0)),
        pl.BlockSpec((d, 1), lambda i, j: (0, 0)),
        pl.BlockSpec((d, d), lambda i, j: (0, 0)),
        pl.BlockSpec((d, d), lambda i, j: (0, 0)),
    ]
    args = [self_v, neigh_v, w, si, sel, w1a, w1b, w2c, w3a, w3b]
    if resid is not None:
        in_specs.append(pl.BlockSpec((1, mc, d), lambda i, j: (i, j, 0)))
        args.append(resid)

    return pl.pallas_call(
        body,
        grid=(b, nblk),
        in_specs=in_specs,
        out_specs=pl.BlockSpec((1, mc, d), lambda i, j: (i, j, 0)),
        out_shape=jax.ShapeDtypeStruct((b, m, d), jnp.float32),
        compiler_params=pltpu.CompilerParams(
            dimension_semantics=("parallel", "parallel")),
    )(*args)


def kernel(inputs, adj, mask_item, item, adj_all, num, embedding,
           a0, a1, a2, a3, gw1, gw2, gw3):
    b, l = inputs.shape
    s = adj_all.shape[1]
    d = embedding.shape[1]

    adj16 = adj_all.reshape(-1, 16)
    num16 = num.reshape(-1, 16)

    t0 = inputs.reshape(-1)
    t0p = _pad_to(t0, NW * 64).reshape(NW, 64)
    a1_rows = _sc_chase(adj16, t0p, s)

    n_t1 = b * l * s
    t1 = a1_rows[:b * l, :s].reshape(-1)
    t1p = _pad_to(t1, NW * 640).reshape(NW, 640)
    a2_rows = _sc_chase(adj16, t1p, s)

    nall = _sc_chase(num16, jnp.concatenate(
        [t0p.reshape(-1), t1p.reshape(-1)]).reshape(NW, 704), s)
    w1n = nall[:b * l, :s].reshape(b, l, s)
    w2n = nall[NW * 64:NW * 64 + n_t1, :s].reshape(b, l * s, s)

    t2 = a2_rows[:n_t1, :s].reshape(-1)
    t2p = t2.reshape(NW, 75, 96)

    hi_idx = jnp.concatenate([t0, item.reshape(-1)])
    hip = _pad_to(hi_idx, NW * 128).reshape(NW, 1, 128)
    ghi = _sc_embed_gather(embedding, hip, nbuf=1)
    h = ghi[:b * l].reshape(b, l, d)
    item_emb = ghi[b * l:2 * b * l].reshape(b, l, d)

    e1 = _sc_embed_gather(embedding, t1.reshape(NW, 5, 120), nbuf=5)
    e1 = e1.reshape(b, l * s, d)

    e2 = _sc_embed_gather(embedding, t2p, nbuf=5)
    e2 = e2.reshape(b, l * s * s, d)

    a_cat = jnp.concatenate([a0, a1, a2, a3], axis=1).T
    maskf = mask_item.astype(jnp.float32).reshape(b, 1, l)
    h_local, si = _tc_local(h, item_emb, adj, maskf, a_cat)

    def wsplit(k):
        return (gw1[k, :d], gw1[k, d:].reshape(1, d), gw2[k],
                gw3[k, :d], gw3[k, d:])

    w1a0, w1b0, w2c0, w3a0, w3b0 = wsplit(0)
    w1a1, w1b1, w2c1, w3a1, w3b1 = wsplit(1)
    mc1 = 200
    eye = jnp.eye(mc1, dtype=jnp.float32)
    sel120 = jnp.repeat(eye, s, axis=1)
    sel50 = sel120[:l, :l * s]
    wf1 = w1n.reshape(b, l * s, 1)
    wf2 = w2n.reshape(b, l * s * s, 1)

    v0 = _tc_global(h, e1, wf1, si, sel50,
                    w1a0, w1b0, w2c0, w3a0, w3b0, mc=l, s=s)
    v1 = _tc_global(e1, e2, wf2, si, sel120,
                    w1a0, w1b0, w2c0, w3a0, w3b0, mc=mc1, s=s)
    out = _tc_global(v0, v1, wf1, si, sel50,
                     w1a1, w1b1, w2c1, w3a1, w3b1, mc=l, s=s,
                     resid=h_local)
    return out
